# baseline (speedup 1.0000x reference)
import functools

import jax
import jax.numpy as jnp
from jax import lax
from jax.experimental import pallas as pl
from jax.experimental.pallas import tpu as pltpu
from jax.experimental.pallas import tpu_sc as plsc

B, C, H, W = 16, 3, 96, 320
HW = H * W
CHW = C * H * W
CH = C * H
RCH = 46
K = 50
KPAD = 64
NCH = 32
DET_THR = 0.3
NEG = -1e30

NUM_TILES = 32
DPT = (B * KPAD) // NUM_TILES
EPT = DPT * NCH


def _topk_body(hm_ref, scores_ref, inds_ref, idx_ref, mt_ref):
    b = pl.program_id(0)
    x = jax.nn.sigmoid(hm_ref[0])
    neg_row = jnp.full((C, 1, W), NEG, jnp.float32)
    up = jnp.concatenate([x[:, 1:, :], neg_row], axis=1)
    dn = jnp.concatenate([neg_row, x[:, :-1, :]], axis=1)
    m = jnp.maximum(jnp.maximum(up, dn), x)
    neg_col = jnp.full((C, H, 1), NEG, jnp.float32)
    lf = jnp.concatenate([m[:, :, 1:], neg_col], axis=2)
    rt = jnp.concatenate([neg_col, m[:, :, :-1]], axis=2)
    m = jnp.maximum(jnp.maximum(lf, rt), m)
    masked = jnp.where(m == x, x, -1.0).reshape(CH, W)

    S = jnp.max(masked, axis=0, keepdims=True)
    mt_ref[...] = masked.T

    iota_w = lax.broadcasted_iota(jnp.int32, (1, W), 1)
    iota_r = lax.broadcasted_iota(jnp.int32, (1, CH), 1)
    iota_k = lax.broadcasted_iota(jnp.int32, (1, KPAD), 1)

    def body(k, carry):
        S, sc_v, ind_v = carry
        gmax = jnp.max(S)
        colhit = S == gmax
        col0 = jnp.min(jnp.where(colhit, iota_w, W))
        colvec = mt_ref[pl.ds(col0, 1), :]
        rowhit = colvec == gmax
        row0 = jnp.min(jnp.where(rowhit, iota_r, CH))
        nhit = jnp.sum(colhit.astype(jnp.int32)) + jnp.sum(rowhit.astype(jnp.int32))

        def tie_scan():
            mt = mt_ref[...]
            sub_w = lax.broadcasted_iota(jnp.int32, (W, CH), 0)
            lane_r = lax.broadcasted_iota(jnp.int32, (W, CH), 1)
            flat = lane_r * W + sub_w
            return jnp.min(jnp.where(mt == gmax, flat, CHW))

        ind = lax.cond(nhit == 2, lambda: row0 * W + col0, tie_scan)
        col = lax.rem(ind, W)
        row = lax.div(ind, W)
        colv = mt_ref[pl.ds(col, 1), :]
        newcol = jnp.where(iota_r == row, -2.0, colv)
        mt_ref[pl.ds(col, 1), :] = newcol
        S = jnp.where(iota_w == col, jnp.max(newcol), S)
        sc_v = jnp.where(iota_k == k, gmax, sc_v)
        ind_v = jnp.where(iota_k == k, ind, ind_v)
        return S, sc_v, ind_v

    init = (S, jnp.full((1, KPAD), NEG, jnp.float32),
            jnp.zeros((1, KPAD), jnp.int32))
    _, sc_v, ind_v = lax.fori_loop(0, K, body, init)

    scores_ref[...] = sc_v[None]
    inds_ref[...] = ind_v[None]
    base = b * (RCH * HW) + (ind_v % HW)
    pad_base = iota_k * 64 + b * 4096
    base = jnp.where(iota_k < K, base, pad_base)
    base_col = base.reshape(KPAD, 1)
    iota_c = lax.broadcasted_iota(jnp.int32, (KPAD, NCH), 1)
    idx_ref[...] = (base_col + iota_c * HW)[None]


def _topk_call(hm):
    out_shapes = (
        jax.ShapeDtypeStruct((B, 1, KPAD), jnp.float32),
        jax.ShapeDtypeStruct((B, 1, KPAD), jnp.int32),
        jax.ShapeDtypeStruct((B, KPAD, NCH), jnp.int32),
    )
    return pl.pallas_call(
        _topk_body,
        grid=(B,),
        in_specs=[pl.BlockSpec((1, C, H, W), lambda b: (b, 0, 0, 0))],
        out_specs=(
            pl.BlockSpec((1, 1, KPAD), lambda b: (b, 0, 0)),
            pl.BlockSpec((1, 1, KPAD), lambda b: (b, 0, 0)),
            pl.BlockSpec((1, KPAD, NCH), lambda b: (b, 0, 0)),
        ),
        out_shape=out_shapes,
        scratch_shapes=[pltpu.VMEM((W, CH), jnp.float32)],
    )(hm)


def _sc_gather_body(idx_hbm, reg_hbm, out_hbm, idx_v, rows_v, sem):
    wid = lax.axis_index("s") * 2 + lax.axis_index("c")
    rpt = EPT // 128
    pltpu.sync_copy(idx_hbm.at[pl.ds(wid * rpt, rpt)], idx_v)

    copies = [
        pltpu.async_copy(reg_hbm.at[idx_v.at[q]],
                         rows_v.at[pl.ds(q * 128, 128)], sem)
        for q in range(EPT // 128)
    ]
    for cp in copies:
        cp.wait()
    pltpu.sync_copy(rows_v, out_hbm.at[pl.ds(wid * EPT, EPT)])


def _sc_gather_call(idx2d, reg_flat):
    mesh = plsc.VectorSubcoreMesh(core_axis_name="c", subcore_axis_name="s")
    f = functools.partial(
        pl.kernel,
        mesh=mesh,
        out_type=jax.ShapeDtypeStruct((NUM_TILES * EPT,), jnp.float32),
        scratch_types=[
            pltpu.VMEM((EPT // 128, 128), jnp.int32),
            pltpu.VMEM((EPT,), jnp.float32),
            pltpu.SemaphoreType.DMA,
        ],
    )(_sc_gather_body)
    return f(idx2d, reg_flat)


def _asm_body(sc_ref, ind_ref, g_ref, out_ref):
    sc = sc_ref[...]
    ind = ind_ref[...]
    g = g_ref[...]
    spatial = ind % HW
    cls = (ind // HW).astype(jnp.float32)
    ysf = (spatial // W).astype(jnp.float32)
    xsf = (spatial % W).astype(jnp.float32)
    valid = (sc >= DET_THR).astype(jnp.float32)
    cols = jnp.concatenate(
        [xsf, ysf, jnp.maximum(g[:, 0:4], 0.0), g[:, 4:25], sc, cls], axis=1)
    out_ref[...] = cols * valid


def _asm_call(sc, ind, g):
    return pl.pallas_call(
        _asm_body,
        out_shape=jax.ShapeDtypeStruct((B * K, 29), jnp.float32),
    )(sc, ind, g)


def kernel(pred_heatmap, pred_regression):
    scores, inds, idx3 = _topk_call(pred_heatmap)
    reg_flat = pred_regression.reshape(-1)
    gout = _sc_gather_call(idx3.reshape(-1, 128), reg_flat)
    g800 = gout.reshape(B, KPAD, NCH)[:, :K].reshape(B * K, NCH)
    sc800 = scores.reshape(B, KPAD)[:, :K].reshape(B * K, 1)
    ind800 = inds.reshape(B, KPAD)[:, :K].reshape(B * K, 1)
    return _asm_call(sc800, ind800, g800)


# speedup vs baseline: 3.6496x; 3.6496x over previous
import functools

import jax
import jax.numpy as jnp
from jax import lax
from jax.experimental import pallas as pl
from jax.experimental.pallas import tpu as pltpu
from jax.experimental.pallas import tpu_sc as plsc

B, C, H, W = 16, 3, 96, 320
HW = H * W
CHW = C * H * W
CH = C * H
RCH = 46
K = 50
KPAD = 64
NCH = 32
DET_THR = 0.3
NEG = -1e30

NUM_TILES = 32
DPT = (B * KPAD) // NUM_TILES
EPT = DPT * NCH


NLEV = 4
NSLOT = 1024
NRG = CH // 8


def _topk_body(hm_ref, scores_ref, inds_ref, idx_ref, vv_ref, vi_ref, d_ref):
    pid = pl.program_id(0)

    @pl.when(pid < B)
    def phase_a():
        b = pid
        x = jax.nn.sigmoid(hm_ref[0])
        neg_row = jnp.full((C, 1, W), NEG, jnp.float32)
        up = jnp.concatenate([x[:, 1:, :], neg_row], axis=1)
        dn = jnp.concatenate([neg_row, x[:, :-1, :]], axis=1)
        m = jnp.maximum(jnp.maximum(up, dn), x)
        neg_col = jnp.full((C, H, 1), NEG, jnp.float32)
        lf = jnp.concatenate([m[:, :, 1:], neg_col], axis=2)
        rt = jnp.concatenate([neg_col, m[:, :, :-1]], axis=2)
        m = jnp.maximum(jnp.maximum(lf, rt), m)
        masked = jnp.where(m == x, x, -1.0).reshape(CH, W)
        d_ref[b] = masked
        mpad = jnp.concatenate(
            [masked, jnp.full((CH, 384 - W), -3.0, jnp.float32)], axis=1)

        base8 = (lax.broadcasted_iota(jnp.int32, (8, 128), 0) * W
                 + lax.broadcasted_iota(jnp.int32, (8, 128), 1))
        lv = [jnp.full((8, 128), -3.0, jnp.float32) for _ in range(NLEV)]
        li = [jnp.zeros((8, 128), jnp.int32) for _ in range(NLEV)]
        for r in range(NRG):
            for part in range(3):
                xv = mpad[8 * r:8 * r + 8, 128 * part:128 * part + 128]
                xi = base8 + (r * 8 * W + part * 128)
                for l in range(NLEV):
                    sw = xv > lv[l]
                    lv[l], xv = (jnp.where(sw, xv, lv[l]),
                                 jnp.where(sw, lv[l], xv))
                    li[l], xi = (jnp.where(sw, xi, li[l]),
                                 jnp.where(sw, li[l], xi))
        vv_ref[b] = jnp.concatenate(lv, axis=0)
        vi_ref[b] = jnp.concatenate(li, axis=0)

    @pl.when(pid == B)
    def phase_b():
        vv0 = vv_ref[...]
        vi0 = vi_ref[...]
        iota_k64 = lax.broadcasted_iota(jnp.int32, (B, KPAD), 1)
        lastlev = lax.broadcasted_iota(jnp.int32, (B, NLEV * 8, 128), 1) >= (NLEV - 1) * 8
        sc_init = jnp.full((B, KPAD), -3.0, jnp.float32)
        ind_init = jnp.zeros((B, KPAD), jnp.int32)

        def ext(k, c):
            vv, sc_a, ind_a, flag = c
            gmax = jnp.max(vv, axis=(1, 2), keepdims=True)
            hit = vv == gmax
            mini = jnp.min(jnp.where(hit, vi0, CHW), axis=(1, 2),
                           keepdims=True)
            kill = hit & (vi0 == mini)
            flag = flag | jnp.any(kill & lastlev).astype(jnp.int32)
            vv = jnp.where(kill, -3.0, vv)
            sc_a = jnp.where(iota_k64 == k, gmax[:, :, 0], sc_a)
            ind_a = jnp.where(iota_k64 == k, mini[:, :, 0], ind_a)
            return vv, sc_a, ind_a, flag

        _, sc_fast, ind_fast, flag = lax.fori_loop(
            0, K, ext, (vv0, sc_init, ind_init, jnp.int32(0)))

        def slow():
            flatio = (lax.broadcasted_iota(jnp.int32, (CH, W), 0) * W
                      + lax.broadcasted_iota(jnp.int32, (CH, W), 1))
            io64 = lax.broadcasted_iota(jnp.int32, (1, KPAD), 1)
            bio = lax.broadcasted_iota(jnp.int32, (B, 1), 0)

            def per_b(b, acc):
                sc_a, ind_a = acc
                dd = d_ref[b]

                def ext2(k, c2):
                    d_, scv, indv = c2
                    g = jnp.max(d_)
                    f = jnp.min(jnp.where(d_ == g, flatio, CHW))
                    d_ = jnp.where(flatio == f, -3.0, d_)
                    scv = jnp.where(io64 == k, g, scv)
                    indv = jnp.where(io64 == k, f, indv)
                    return d_, scv, indv

                _, scv, indv = lax.fori_loop(
                    0, K, ext2,
                    (dd, jnp.full((1, KPAD), -3.0, jnp.float32),
                     jnp.zeros((1, KPAD), jnp.int32)))
                rowm = bio == b
                sc_a = jnp.where(rowm, scv, sc_a)
                ind_a = jnp.where(rowm, indv, ind_a)
                return sc_a, ind_a

            return lax.fori_loop(0, B, per_b, (sc_init, ind_init))

        sc_all, ind_all = lax.cond(flag != 0, slow,
                                   lambda: (sc_fast, ind_fast))

        scores_ref[...] = sc_all[:, None, :]
        inds_ref[...] = ind_all[:, None, :]
        b_col = lax.broadcasted_iota(jnp.int32, (B, KPAD), 0)
        base = b_col * (RCH * HW) + (ind_all % HW)
        pad_base = iota_k64 * 64 + b_col * 4096
        base = jnp.where(iota_k64 < K, base, pad_base)
        c3 = lax.broadcasted_iota(jnp.int32, (B, KPAD, NCH), 2)
        idx_ref[...] = base[:, :, None] + c3 * HW


def _topk_call(hm):
    out_shapes = (
        jax.ShapeDtypeStruct((B, 1, KPAD), jnp.float32),
        jax.ShapeDtypeStruct((B, 1, KPAD), jnp.int32),
        jax.ShapeDtypeStruct((B, KPAD, NCH), jnp.int32),
    )
    return pl.pallas_call(
        _topk_body,
        grid=(B + 1,),
        in_specs=[pl.BlockSpec((1, C, H, W),
                               lambda i: (jnp.minimum(i, B - 1), 0, 0, 0))],
        out_specs=(
            pl.BlockSpec((B, 1, KPAD), lambda i: (0, 0, 0)),
            pl.BlockSpec((B, 1, KPAD), lambda i: (0, 0, 0)),
            pl.BlockSpec((B, KPAD, NCH), lambda i: (0, 0, 0)),
        ),
        out_shape=out_shapes,
        scratch_shapes=[
            pltpu.VMEM((B, NLEV * 8, 128), jnp.float32),
            pltpu.VMEM((B, NLEV * 8, 128), jnp.int32),
            pltpu.VMEM((B, CH, W), jnp.float32),
        ],
    )(hm)


def _sc_gather_body(idx_hbm, reg_hbm, out_hbm, idx_v, rows_v, sem):
    wid = lax.axis_index("s") * 2 + lax.axis_index("c")
    rpt = EPT // 128
    pltpu.sync_copy(idx_hbm.at[pl.ds(wid * rpt, rpt)], idx_v)

    copies = [
        pltpu.async_copy(reg_hbm.at[idx_v.at[q]],
                         rows_v.at[pl.ds(q * 128, 128)], sem)
        for q in range(EPT // 128)
    ]
    for cp in copies:
        cp.wait()
    pltpu.sync_copy(rows_v, out_hbm.at[pl.ds(wid * EPT, EPT)])


def _sc_gather_call(idx2d, reg_flat):
    mesh = plsc.VectorSubcoreMesh(core_axis_name="c", subcore_axis_name="s")
    f = functools.partial(
        pl.kernel,
        mesh=mesh,
        out_type=jax.ShapeDtypeStruct((NUM_TILES * EPT,), jnp.float32),
        scratch_types=[
            pltpu.VMEM((EPT // 128, 128), jnp.int32),
            pltpu.VMEM((EPT,), jnp.float32),
            pltpu.SemaphoreType.DMA,
        ],
    )(_sc_gather_body)
    return f(idx2d, reg_flat)


def _asm_body(sc_ref, ind_ref, g_ref, out_ref):
    sc = sc_ref[...]
    ind = ind_ref[...]
    g = g_ref[...]
    spatial = ind % HW
    cls = (ind // HW).astype(jnp.float32)
    ysf = (spatial // W).astype(jnp.float32)
    xsf = (spatial % W).astype(jnp.float32)
    valid = (sc >= DET_THR).astype(jnp.float32)
    cols = jnp.concatenate(
        [xsf, ysf, jnp.maximum(g[:, 0:4], 0.0), g[:, 4:25], sc, cls], axis=1)
    out_ref[...] = cols * valid


def _asm_call(sc, ind, g):
    return pl.pallas_call(
        _asm_body,
        out_shape=jax.ShapeDtypeStruct((B * K, 29), jnp.float32),
    )(sc, ind, g)


def kernel(pred_heatmap, pred_regression):
    scores, inds, idx3 = _topk_call(pred_heatmap)
    reg_flat = pred_regression.reshape(-1)
    gout = _sc_gather_call(idx3.reshape(-1, 128), reg_flat)
    g800 = gout.reshape(B, KPAD, NCH)[:, :K].reshape(B * K, NCH)
    sc800 = scores.reshape(B, KPAD)[:, :K].reshape(B * K, 1)
    ind800 = inds.reshape(B, KPAD)[:, :K].reshape(B * K, 1)
    return _asm_call(sc800, ind800, g800)


# speedup vs baseline: 3.6660x; 1.0045x over previous
import functools

import jax
import jax.numpy as jnp
from jax import lax
from jax.experimental import pallas as pl
from jax.experimental.pallas import tpu as pltpu
from jax.experimental.pallas import tpu_sc as plsc

B, C, H, W = 16, 3, 96, 320
HW = H * W
CHW = C * H * W
CH = C * H
RCH = 46
K = 50
KPAD = 64
NCH = 32
DET_THR = 0.3
NEG = -1e30

NUM_TILES = 32
DPT = (B * KPAD) // NUM_TILES
EPT = DPT * NCH


NLEV = 4
NSLOT = 1024
NRG = CH // 8


def _topk_body(hm_ref, scores_ref, inds_ref, idx_ref, vv_ref, vi_ref, d_ref):
    pid = pl.program_id(0)

    @pl.when(pid < B)
    def phase_a():
        b = pid
        x = jax.nn.sigmoid(hm_ref[0])
        neg_row = jnp.full((C, 1, W), NEG, jnp.float32)
        up = jnp.concatenate([x[:, 1:, :], neg_row], axis=1)
        dn = jnp.concatenate([neg_row, x[:, :-1, :]], axis=1)
        m = jnp.maximum(jnp.maximum(up, dn), x)
        neg_col = jnp.full((C, H, 1), NEG, jnp.float32)
        lf = jnp.concatenate([m[:, :, 1:], neg_col], axis=2)
        rt = jnp.concatenate([neg_col, m[:, :, :-1]], axis=2)
        m = jnp.maximum(jnp.maximum(lf, rt), m)
        masked = jnp.where(m == x, x, -1.0).reshape(CH, W)
        d_ref[b] = masked
        mpad = jnp.concatenate(
            [masked, jnp.full((CH, 384 - W), -3.0, jnp.float32)], axis=1)

        base8 = (lax.broadcasted_iota(jnp.int32, (8, 128), 0) * W
                 + lax.broadcasted_iota(jnp.int32, (8, 128), 1))
        lv = [jnp.full((8, 128), -3.0, jnp.float32) for _ in range(NLEV)]
        li = [jnp.zeros((8, 128), jnp.int32) for _ in range(NLEV)]
        for r in range(NRG):
            for part in range(3):
                xv = mpad[8 * r:8 * r + 8, 128 * part:128 * part + 128]
                xi = base8 + (r * 8 * W + part * 128)
                for l in range(NLEV):
                    sw = xv > lv[l]
                    lv[l], xv = (jnp.where(sw, xv, lv[l]),
                                 jnp.where(sw, lv[l], xv))
                    li[l], xi = (jnp.where(sw, xi, li[l]),
                                 jnp.where(sw, li[l], xi))
        vv_ref[b] = jnp.concatenate(lv, axis=0)
        vi_ref[b] = jnp.concatenate(li, axis=0)

    @pl.when(pid == B)
    def phase_b():
        vv0 = vv_ref[...]
        vi0 = vi_ref[...]
        iota_k64 = lax.broadcasted_iota(jnp.int32, (B, KPAD), 1)
        lastlev = lax.broadcasted_iota(jnp.int32, (B, NLEV * 8, 128), 1) >= (NLEV - 1) * 8
        sc_init = jnp.full((B, KPAD), -3.0, jnp.float32)
        ind_init = jnp.zeros((B, KPAD), jnp.int32)

        def ext(k, c):
            vv, sc_a, ind_a, flag = c
            gmax = jnp.max(vv, axis=(1, 2), keepdims=True)
            hit = vv == gmax
            mini = jnp.min(jnp.where(hit, vi0, CHW), axis=(1, 2),
                           keepdims=True)
            kill = hit & (vi0 == mini)
            flag = flag | jnp.any(kill & lastlev).astype(jnp.int32)
            vv = jnp.where(kill, -3.0, vv)
            sc_a = jnp.where(iota_k64 == k, gmax[:, :, 0], sc_a)
            ind_a = jnp.where(iota_k64 == k, mini[:, :, 0], ind_a)
            return vv, sc_a, ind_a, flag

        _, sc_fast, ind_fast, flag = lax.fori_loop(
            0, K, ext, (vv0, sc_init, ind_init, jnp.int32(0)))

        def emit(sc_all, ind_all):
            scores_ref[...] = sc_all[:, None, :]
            inds_ref[...] = ind_all[:, None, :]
            b_col = lax.broadcasted_iota(jnp.int32, (B, KPAD), 0)
            base = b_col * (RCH * HW) + (ind_all % HW)
            pad_base = iota_k64 * 64 + b_col * 4096
            base = jnp.where(iota_k64 < K, base, pad_base)
            c3 = lax.broadcasted_iota(jnp.int32, (B, KPAD, NCH), 2)
            idx_ref[...] = base[:, :, None] + c3 * HW

        emit(sc_fast, ind_fast)

        @pl.when(flag != 0)
        def rare_exact_path():
            flatio = (lax.broadcasted_iota(jnp.int32, (CH, W), 0) * W
                      + lax.broadcasted_iota(jnp.int32, (CH, W), 1))
            io64 = lax.broadcasted_iota(jnp.int32, (1, KPAD), 1)
            bio = lax.broadcasted_iota(jnp.int32, (B, 1), 0)

            def per_b(b, acc):
                sc_a, ind_a = acc
                dd = d_ref[b]

                def ext2(k, c2):
                    d_, scv, indv = c2
                    g = jnp.max(d_)
                    f = jnp.min(jnp.where(d_ == g, flatio, CHW))
                    d_ = jnp.where(flatio == f, -3.0, d_)
                    scv = jnp.where(io64 == k, g, scv)
                    indv = jnp.where(io64 == k, f, indv)
                    return d_, scv, indv

                _, scv, indv = lax.fori_loop(
                    0, K, ext2,
                    (dd, jnp.full((1, KPAD), -3.0, jnp.float32),
                     jnp.zeros((1, KPAD), jnp.int32)))
                rowm = bio == b
                sc_a = jnp.where(rowm, scv, sc_a)
                ind_a = jnp.where(rowm, indv, ind_a)
                return sc_a, ind_a

            sc_all, ind_all = lax.fori_loop(0, B, per_b,
                                            (sc_init, ind_init))
            emit(sc_all, ind_all)


def _topk_call(hm):
    out_shapes = (
        jax.ShapeDtypeStruct((B, 1, KPAD), jnp.float32),
        jax.ShapeDtypeStruct((B, 1, KPAD), jnp.int32),
        jax.ShapeDtypeStruct((B, KPAD, NCH), jnp.int32),
    )
    return pl.pallas_call(
        _topk_body,
        grid=(B + 1,),
        in_specs=[pl.BlockSpec((1, C, H, W),
                               lambda i: (jnp.minimum(i, B - 1), 0, 0, 0))],
        out_specs=(
            pl.BlockSpec((B, 1, KPAD), lambda i: (0, 0, 0)),
            pl.BlockSpec((B, 1, KPAD), lambda i: (0, 0, 0)),
            pl.BlockSpec((B, KPAD, NCH), lambda i: (0, 0, 0)),
        ),
        out_shape=out_shapes,
        scratch_shapes=[
            pltpu.VMEM((B, NLEV * 8, 128), jnp.float32),
            pltpu.VMEM((B, NLEV * 8, 128), jnp.int32),
            pltpu.VMEM((B, CH, W), jnp.float32),
        ],
    )(hm)


def _sc_gather_body(idx_hbm, reg_hbm, out_hbm, idx_v, rows_v, sem):
    wid = lax.axis_index("s") * 2 + lax.axis_index("c")
    rpt = EPT // 128
    pltpu.sync_copy(idx_hbm.at[pl.ds(wid * rpt, rpt)], idx_v)

    copies = [
        pltpu.async_copy(reg_hbm.at[idx_v.at[q]],
                         rows_v.at[pl.ds(q * 128, 128)], sem)
        for q in range(EPT // 128)
    ]
    for cp in copies:
        cp.wait()
    pltpu.sync_copy(rows_v, out_hbm.at[pl.ds(wid * EPT, EPT)])


def _sc_gather_call(idx2d, reg_flat):
    mesh = plsc.VectorSubcoreMesh(core_axis_name="c", subcore_axis_name="s")
    f = functools.partial(
        pl.kernel,
        mesh=mesh,
        out_type=jax.ShapeDtypeStruct((NUM_TILES * EPT,), jnp.float32),
        scratch_types=[
            pltpu.VMEM((EPT // 128, 128), jnp.int32),
            pltpu.VMEM((EPT,), jnp.float32),
            pltpu.SemaphoreType.DMA,
        ],
    )(_sc_gather_body)
    return f(idx2d, reg_flat)


def _asm_body(sc_ref, ind_ref, g_ref, out_ref):
    sc = sc_ref[...]
    ind = ind_ref[...]
    g = g_ref[...]
    spatial = ind % HW
    cls = (ind // HW).astype(jnp.float32)
    ysf = (spatial // W).astype(jnp.float32)
    xsf = (spatial % W).astype(jnp.float32)
    valid = (sc >= DET_THR).astype(jnp.float32)
    cols = jnp.concatenate(
        [xsf, ysf, jnp.maximum(g[:, 0:4], 0.0), g[:, 4:25], sc, cls], axis=1)
    out_ref[...] = cols * valid


def _asm_call(sc, ind, g):
    return pl.pallas_call(
        _asm_body,
        out_shape=jax.ShapeDtypeStruct((B * K, 29), jnp.float32),
    )(sc, ind, g)


def kernel(pred_heatmap, pred_regression):
    scores, inds, idx3 = _topk_call(pred_heatmap)
    reg_flat = pred_regression.reshape(-1)
    gout = _sc_gather_call(idx3.reshape(-1, 128), reg_flat)
    g800 = gout.reshape(B, KPAD, NCH)[:, :K].reshape(B * K, NCH)
    sc800 = scores.reshape(B, KPAD)[:, :K].reshape(B * K, 1)
    ind800 = inds.reshape(B, KPAD)[:, :K].reshape(B * K, 1)
    return _asm_call(sc800, ind800, g800)


# speedup vs baseline: 7.9987x; 2.1819x over previous
import jax
import jax.numpy as jnp
from jax import lax
from jax.experimental import pallas as pl
from jax.experimental.pallas import tpu as pltpu

B, C, H, W = 16, 3, 96, 320
HW = H * W
CHW = C * H * W
CH = C * H
K = 50
KPAD = 64
DET_THR = 0.3
NEG = -1e30
NLEV = 3
NRG = CH // 8


def _topk_body(hm_ref, scores_ref, inds_ref, vv_ref, vi_ref, d_ref):
    pid = pl.program_id(0)

    @pl.when(pid < B)
    def phase_a():
        b = pid
        x = jax.nn.sigmoid(hm_ref[0])
        neg_row = jnp.full((C, 1, W), NEG, jnp.float32)
        up = jnp.concatenate([x[:, 1:, :], neg_row], axis=1)
        dn = jnp.concatenate([neg_row, x[:, :-1, :]], axis=1)
        m = jnp.maximum(jnp.maximum(up, dn), x)
        neg_col = jnp.full((C, H, 1), NEG, jnp.float32)
        lf = jnp.concatenate([m[:, :, 1:], neg_col], axis=2)
        rt = jnp.concatenate([neg_col, m[:, :, :-1]], axis=2)
        m = jnp.maximum(jnp.maximum(lf, rt), m)
        masked = jnp.where(m == x, x, -1.0).reshape(CH, W)
        d_ref[b] = masked
        mpad = jnp.concatenate(
            [masked, jnp.full((CH, 384 - W), -3.0, jnp.float32)], axis=1)

        base8 = (lax.broadcasted_iota(jnp.int32, (8, 128), 0) * W
                 + lax.broadcasted_iota(jnp.int32, (8, 128), 1))
        lv = [jnp.full((8, 128), -3.0, jnp.float32) for _ in range(NLEV)]
        li = [jnp.zeros((8, 128), jnp.int32) for _ in range(NLEV)]
        for r in range(NRG):
            for part in range(3):
                xv = mpad[8 * r:8 * r + 8, 128 * part:128 * part + 128]
                xi = base8 + (r * 8 * W + part * 128)
                for l in range(NLEV):
                    sw = xv > lv[l]
                    if l < NLEV - 1:
                        lv[l], xv = (jnp.where(sw, xv, lv[l]),
                                     jnp.where(sw, lv[l], xv))
                        li[l], xi = (jnp.where(sw, xi, li[l]),
                                     jnp.where(sw, li[l], xi))
                    else:
                        lv[l] = jnp.where(sw, xv, lv[l])
                        li[l] = jnp.where(sw, xi, li[l])
        vv_ref[b] = jnp.concatenate(lv, axis=0)
        vi_ref[b] = jnp.concatenate(li, axis=0)

    @pl.when(pid == B)
    def phase_b():
        vv0 = vv_ref[...]
        vi0 = vi_ref[...]
        iota_k64 = lax.broadcasted_iota(jnp.int32, (B, KPAD), 1)
        sc_init = jnp.full((B, KPAD), -3.0, jnp.float32)
        ind_init = jnp.zeros((B, KPAD), jnp.int32)

        def ext(k, c):
            l1v, l1i, l2v, l2i, l3v, kcnt, sc_a, ind_a, flag = c
            gmax = jnp.max(l1v, axis=(1, 2), keepdims=True)
            hit = l1v == gmax
            mini = jnp.min(jnp.where(hit, l1i, CHW), axis=(1, 2),
                           keepdims=True)
            kill = hit & (l1i == mini)
            flag = flag | jnp.any(kill & (kcnt == NLEV - 1)).astype(jnp.int32)
            kcnt = kcnt + kill.astype(jnp.int32)
            l1v = jnp.where(kill, l2v, l1v)
            l1i = jnp.where(kill, l2i, l1i)
            l2v = jnp.where(kill, l3v, l2v)
            l2i = jnp.where(kill, vi0[:, 16:24], l2i)
            l3v = jnp.where(kill, -3.0, l3v)
            sc_a = jnp.where(iota_k64 == k, gmax[:, :, 0], sc_a)
            ind_a = jnp.where(iota_k64 == k, mini[:, :, 0], ind_a)
            return l1v, l1i, l2v, l2i, l3v, kcnt, sc_a, ind_a, flag

        init = (vv0[:, 0:8], vi0[:, 0:8], vv0[:, 8:16], vi0[:, 8:16],
                vv0[:, 16:24], jnp.zeros((B, 8, 128), jnp.int32),
                sc_init, ind_init, jnp.int32(0))
        out = lax.fori_loop(0, K, ext, init)
        sc_fast, ind_fast, flag = out[6], out[7], out[8]

        def emit(sc_all, ind_all):
            scores_ref[...] = sc_all[:, None, :]
            inds_ref[...] = ind_all[:, None, :]

        emit(sc_fast, ind_fast)

        @pl.when(flag != 0)
        def rare_exact_path():
            flatio = (lax.broadcasted_iota(jnp.int32, (CH, W), 0) * W
                      + lax.broadcasted_iota(jnp.int32, (CH, W), 1))
            io64 = lax.broadcasted_iota(jnp.int32, (1, KPAD), 1)
            bio = lax.broadcasted_iota(jnp.int32, (B, 1), 0)

            def per_b(b, acc):
                sc_a, ind_a = acc
                dd = d_ref[b]

                def ext2(k, c2):
                    d_, scv, indv = c2
                    g = jnp.max(d_)
                    f = jnp.min(jnp.where(d_ == g, flatio, CHW))
                    d_ = jnp.where(flatio == f, -3.0, d_)
                    scv = jnp.where(io64 == k, g, scv)
                    indv = jnp.where(io64 == k, f, indv)
                    return d_, scv, indv

                _, scv, indv = lax.fori_loop(
                    0, K, ext2,
                    (dd, jnp.full((1, KPAD), -3.0, jnp.float32),
                     jnp.zeros((1, KPAD), jnp.int32)))
                rowm = bio == b
                sc_a = jnp.where(rowm, scv, sc_a)
                ind_a = jnp.where(rowm, indv, ind_a)
                return sc_a, ind_a

            sc_all, ind_all = lax.fori_loop(0, B, per_b,
                                            (sc_init, ind_init))
            emit(sc_all, ind_all)


def _topk_call(hm):
    out_shapes = (
        jax.ShapeDtypeStruct((B, 1, KPAD), jnp.float32),
        jax.ShapeDtypeStruct((B, 1, KPAD), jnp.int32),
    )
    return pl.pallas_call(
        _topk_body,
        grid=(B + 1,),
        in_specs=[pl.BlockSpec((1, C, H, W),
                               lambda i: (jnp.minimum(i, B - 1), 0, 0, 0))],
        out_specs=(
            pl.BlockSpec((B, 1, KPAD), lambda i: (0, 0, 0)),
            pl.BlockSpec((B, 1, KPAD), lambda i: (0, 0, 0)),
        ),
        out_shape=out_shapes,
        scratch_shapes=[
            pltpu.VMEM((B, NLEV * 8, 128), jnp.float32),
            pltpu.VMEM((B, NLEV * 8, 128), jnp.int32),
            pltpu.VMEM((B, CH, W), jnp.float32),
        ],
    )(hm)


GCH = 25
HP = jax.lax.Precision.HIGHEST


def _gather_asm_body(sc_ref, ind_ref, reg_ref, out_ref):
    sc = sc_ref[0]
    ind = ind_ref[0]
    spatial = ind % HW
    ys = spatial // W
    xs = spatial % W
    cls = ind // HW
    meta = jnp.concatenate(
        [xs.astype(jnp.float32), ys.astype(jnp.float32),
         cls.astype(jnp.float32), sc], axis=0)
    eye = (lax.broadcasted_iota(jnp.int32, (KPAD, KPAD), 0)
           == lax.broadcasted_iota(jnp.int32, (KPAD, KPAD), 1)).astype(jnp.float32)
    dn = (((1,), (1,)), ((), ()))
    metat = jax.lax.dot_general(eye, meta, dn, precision=HP,
                                preferred_element_type=jnp.float32)
    u2 = (lax.broadcasted_iota(jnp.int32, (H, KPAD), 0) == ys).astype(jnp.float32)
    xsti = metat[:, 0:1].astype(jnp.int32)
    v2t = (lax.broadcasted_iota(jnp.int32, (KPAD, W), 1)
           == xsti).astype(jnp.float32)
    r = reg_ref[0]
    dny = (((0,), (0,)), ((), ()))
    pois_cols = []
    for c in range(GCH):
        s_c = jax.lax.dot_general(u2, r[c], dny, precision=HP,
                                  preferred_element_type=jnp.float32)
        pois_cols.append(jnp.sum(s_c * v2t, axis=1, keepdims=True))
    poist = jnp.concatenate(pois_cols, axis=1)
    valid = (metat[:, 3:4] >= DET_THR).astype(jnp.float32)
    cols = jnp.concatenate(
        [metat[:, 0:2], jnp.maximum(poist[:, 0:4], 0.0), poist[:, 4:25],
         metat[:, 3:4], metat[:, 2:3]], axis=1)
    out_ref[0] = (cols * valid)[:K]


def _gather_asm_call(scores, inds, reg):
    return pl.pallas_call(
        _gather_asm_body,
        grid=(B,),
        in_specs=[
            pl.BlockSpec((1, 1, KPAD), lambda b: (b, 0, 0)),
            pl.BlockSpec((1, 1, KPAD), lambda b: (b, 0, 0)),
            pl.BlockSpec((1, GCH, H, W), lambda b: (b, 0, 0, 0)),
        ],
        out_specs=pl.BlockSpec((1, K, 29), lambda b: (b, 0, 0)),
        out_shape=jax.ShapeDtypeStruct((B, K, 29), jnp.float32),
    )(scores, inds, reg)


def kernel(pred_heatmap, pred_regression):
    scores, inds = _topk_call(pred_heatmap)
    res = _gather_asm_call(scores, inds, pred_regression)
    return res.reshape(B * K, 29)
